# baseline (speedup 1.0000x reference)
import functools

import jax
import jax.numpy as jnp
from jax import lax
from jax.experimental import pallas as pl
from jax.experimental.pallas import tpu as pltpu
from jax.experimental.pallas import tpu_sc as plsc

VOCAB = 1000000
DIM = 64
BATCH = 16384
HIST = 200
M = BATCH * HIST
NC, NS = 2, 16
NW = NC * NS
B_W = M // NW
K = 128
NBLK = B_W // K

_mesh = plsc.VectorSubcoreMesh(core_axis_name="c", subcore_axis_name="s")


@functools.partial(
    pl.kernel,
    mesh=_mesh,
    compiler_params=pltpu.CompilerParams(use_tc_tiling_on_sc=False),
    out_type=jax.ShapeDtypeStruct((M, DIM), jnp.float32),
    scratch_types=[
        pltpu.VMEM((K,), jnp.int32),
        pltpu.VMEM((K, DIM), jnp.float32),
        pltpu.SemaphoreType.DMA,
    ],
)
def _emb_gather(x_hbm, table_hbm, out_hbm, idx_v, rows_v, sem):
    wid = lax.axis_index("s") * NC + lax.axis_index("c")
    base = wid * B_W

    def body(k, carry):
        pos = base + k * K
        pltpu.sync_copy(x_hbm.at[pl.ds(pos, K)], idx_v)
        pltpu.async_copy(table_hbm.at[idx_v], rows_v, sem).wait()
        pltpu.sync_copy(rows_v, out_hbm.at[pl.ds(pos, K)])
        return carry

    lax.fori_loop(0, NBLK, body, 0)


def kernel(x, table):
    xf = x.astype(jnp.int32).reshape(M)
    out = _emb_gather(xf, table)
    return out.reshape(BATCH, HIST, DIM)


# speedup vs baseline: 1.3021x; 1.3021x over previous
import functools

import jax
import jax.numpy as jnp
from jax import lax
from jax.experimental import pallas as pl
from jax.experimental.pallas import tpu as pltpu
from jax.experimental.pallas import tpu_sc as plsc

VOCAB = 1000000
DIM = 64
BATCH = 16384
HIST = 200
M = BATCH * HIST
IW = 128
MR = M // IW
NC, NS = 2, 16
NW = NC * NS
R = 4
SB = R * IW
NSB = M // (SB * NW)
NBUF = 2
ROWS_W = MR // NW

_mesh = plsc.VectorSubcoreMesh(core_axis_name="c", subcore_axis_name="s")


@functools.partial(
    pl.kernel,
    mesh=_mesh,
    compiler_params=pltpu.CompilerParams(use_tc_tiling_on_sc=False),
    out_type=jax.ShapeDtypeStruct((M, DIM), jnp.float32),
    scratch_types=[
        pltpu.VMEM((NBUF, R, IW), jnp.int32),
        pltpu.VMEM((NBUF, SB, DIM), jnp.float32),
        pltpu.SemaphoreType.DMA,
        pltpu.SemaphoreType.DMA,
        pltpu.SemaphoreType.DMA,
        pltpu.SemaphoreType.DMA,
    ],
)
def _emb_gather(x_hbm, table_hbm, out_hbm, idx_v, rows_v, g0, g1, w0, w1):
    wid = lax.axis_index("s") * NC + lax.axis_index("c")
    row0 = wid * ROWS_W
    gsem = (g0, g1)
    wsem = (w0, w1)

    def stage_idx(sb, b):
        pltpu.sync_copy(x_hbm.at[pl.ds(row0 + sb * R, R)], idx_v.at[b])

    def fire_gathers(b):
        for j in range(R):
            pltpu.async_copy(
                table_hbm.at[idx_v.at[b, j]],
                rows_v.at[b, pl.ds(j * IW, IW)],
                gsem[b],
            )

    def drain_gathers(b):
        pltpu.make_async_copy(
            table_hbm.at[pl.ds(0, SB)], rows_v.at[b], gsem[b]
        ).wait()

    for b in range(NBUF):
        stage_idx(b, b)
        fire_gathers(b)

    def body(g, carry):
        wbs = []
        for b in range(NBUF):
            drain_gathers(b)
            out_pos = (row0 + (g * NBUF + b) * R) * IW
            wbs.append(
                pltpu.async_copy(
                    rows_v.at[b], out_hbm.at[pl.ds(out_pos, SB)], wsem[b]
                )
            )
        for b in range(NBUF):
            stage_idx(g * NBUF + b + NBUF, b)
        for b in range(NBUF):
            wbs[b].wait()
            fire_gathers(b)
        return carry

    lax.fori_loop(0, (NSB - NBUF) // NBUF, body, 0)

    for b in range(NBUF):
        drain_gathers(b)
        out_pos = (row0 + (NSB - NBUF + b) * R) * IW
        pltpu.sync_copy(rows_v.at[b], out_hbm.at[pl.ds(out_pos, SB)])


def kernel(x, table):
    xf = x.astype(jnp.int32).reshape(MR, IW)
    out = _emb_gather(xf, table)
    return out.reshape(BATCH, HIST, DIM)


# speedup vs baseline: 1.3188x; 1.0128x over previous
import functools

import jax
import jax.numpy as jnp
from jax import lax
from jax.experimental import pallas as pl
from jax.experimental.pallas import tpu as pltpu
from jax.experimental.pallas import tpu_sc as plsc

VOCAB = 1000000
DIM = 64
BATCH = 16384
HIST = 200
M = BATCH * HIST
IW = 128
MR = M // IW
NC, NS = 2, 16
NW = NC * NS
R = 4
SB = R * IW
NSB = M // (SB * NW)
NBUF = 3
ROWS_W = MR // NW
NMAIN = (NSB - 5) // NBUF

_mesh = plsc.VectorSubcoreMesh(core_axis_name="c", subcore_axis_name="s")


@functools.partial(
    pl.kernel,
    mesh=_mesh,
    compiler_params=pltpu.CompilerParams(use_tc_tiling_on_sc=False),
    out_type=jax.ShapeDtypeStruct((M, DIM), jnp.float32),
    scratch_types=[
        pltpu.VMEM((NBUF, R, IW), jnp.int32),
        pltpu.VMEM((NBUF, SB, DIM), jnp.float32),
        pltpu.SemaphoreType.DMA,
        pltpu.SemaphoreType.DMA,
        pltpu.SemaphoreType.DMA,
        pltpu.SemaphoreType.DMA,
        pltpu.SemaphoreType.DMA,
        pltpu.SemaphoreType.DMA,
    ],
)
def _emb_gather(x_hbm, table_hbm, out_hbm, idx_v, rows_v, g0, g1, g2, w0, w1, w2):
    wid = lax.axis_index("s") * NC + lax.axis_index("c")
    row0 = wid * ROWS_W
    gsem = (g0, g1, g2)
    wsem = (w0, w1, w2)

    def stage_idx(sb, b):
        pltpu.sync_copy(x_hbm.at[pl.ds(row0 + sb * R, R)], idx_v.at[b])

    def fire_gathers(b):
        for j in range(R):
            pltpu.async_copy(
                table_hbm.at[idx_v.at[b, j]],
                rows_v.at[b, pl.ds(j * IW, IW)],
                gsem[b],
            )

    def drain_gathers(b):
        pltpu.make_async_copy(
            table_hbm.at[pl.ds(0, SB)], rows_v.at[b], gsem[b]
        ).wait()

    for b in range(NBUF):
        stage_idx(b, b)
        fire_gathers(b)

    def body(g, carry):
        for b in range(NBUF):
            sb0 = g * NBUF + b
            drain_gathers(b)
            out_pos = (row0 + sb0 * R) * IW
            wb = pltpu.async_copy(
                rows_v.at[b], out_hbm.at[pl.ds(out_pos, SB)], wsem[b]
            )
            stage_idx(sb0 + NBUF, b)
            wb.wait()
            fire_gathers(b)
        return carry

    lax.fori_loop(0, NMAIN, body, 0)

    for b in range(NBUF):
        sb = NMAIN * NBUF + b
        drain_gathers(b)
        out_pos = (row0 + sb * R) * IW
        pltpu.sync_copy(rows_v.at[b], out_hbm.at[pl.ds(out_pos, SB)])
    for b in range(NSB - NMAIN * NBUF - NBUF):
        sb = NMAIN * NBUF + NBUF + b
        stage_idx(sb, b)
        fire_gathers(b)
        drain_gathers(b)
        out_pos = (row0 + sb * R) * IW
        pltpu.sync_copy(rows_v.at[b], out_hbm.at[pl.ds(out_pos, SB)])


def kernel(x, table):
    xf = x.astype(jnp.int32).reshape(MR, IW)
    out = _emb_gather(xf, table)
    return out.reshape(BATCH, HIST, DIM)
